# baseline (speedup 1.0000x reference)
import jax
import jax.numpy as jnp
from jax.experimental import pallas as pl
from jax.experimental.pallas import tpu as pltpu

CH = 1024
NBUF = 6


def _router_kernel(x_hbm, w_ref, b_ref, logits_hbm, probs_hbm,
                   buf, logits_v, probs_v, insems, outsem):
    tokens = x_hbm.shape[0]
    nchunks = tokens // CH

    def copy_in(t, slot):
        return pltpu.make_async_copy(
            x_hbm.at[pl.ds(t * CH, CH), :],
            buf.at[slot],
            insems.at[slot],
        )

    for k in range(NBUF):
        copy_in(k, k).start()

    def body(t, carry):
        slot = jax.lax.rem(t, NBUF)
        copy_in(t, slot).wait()
        logits = jnp.dot(buf[slot], w_ref[...],
                         preferred_element_type=jnp.float32)
        logits = logits + b_ref[...]
        rows = pl.ds(t * CH, CH)
        logits_v[rows, :] = logits
        m = jnp.max(logits, axis=-1, keepdims=True)
        e = jnp.exp(logits - m)
        probs_v[rows, :] = e / jnp.sum(e, axis=-1, keepdims=True)
        pltpu.make_async_copy(
            logits_v.at[rows, :], logits_hbm.at[rows, :], outsem.at[0]
        ).start()
        pltpu.make_async_copy(
            probs_v.at[rows, :], probs_hbm.at[rows, :], outsem.at[1]
        ).start()

        @pl.when(t + NBUF < nchunks)
        def _prefetch():
            copy_in(t + NBUF, jax.lax.rem(t + NBUF, NBUF)).start()

        return carry

    jax.lax.fori_loop(0, nchunks, body, 0)

    for t in range(nchunks):
        rows = pl.ds(t * CH, CH)
        pltpu.make_async_copy(
            logits_v.at[rows, :], logits_hbm.at[rows, :], outsem.at[0]
        ).wait()
        pltpu.make_async_copy(
            probs_v.at[rows, :], probs_hbm.at[rows, :], outsem.at[1]
        ).wait()


def kernel(x, W, b):
    tokens, d = x.shape
    na = W.shape[1]
    b2 = b.reshape(1, na)
    out_shape = jax.ShapeDtypeStruct((tokens, na), jnp.float32)
    logits, probs = pl.pallas_call(
        _router_kernel,
        grid=(1,),
        in_specs=[
            pl.BlockSpec(memory_space=pltpu.HBM),
            pl.BlockSpec((d, na), lambda i: (0, 0)),
            pl.BlockSpec((1, na), lambda i: (0, 0)),
        ],
        out_specs=[
            pl.BlockSpec(memory_space=pltpu.HBM),
            pl.BlockSpec(memory_space=pltpu.HBM),
        ],
        out_shape=[out_shape, out_shape],
        scratch_shapes=[
            pltpu.VMEM((NBUF, CH, d), jnp.float32),
            pltpu.VMEM((tokens, na), jnp.float32),
            pltpu.VMEM((tokens, na), jnp.float32),
            pltpu.SemaphoreType.DMA((NBUF,)),
            pltpu.SemaphoreType.DMA((2,)),
        ],
        compiler_params=pltpu.CompilerParams(
            dimension_semantics=(pltpu.ARBITRARY,),
            vmem_limit_bytes=100 * 1024 * 1024,
            skip_device_barrier=True,
        ),
    )(x, W, b2)
    return (logits, probs)
